# baseline (speedup 1.0000x reference)
import jax
import jax.numpy as jnp
from jax import lax
from jax.experimental import pallas as pl

_N_E = 8192
_E_DIM = 32
_BETA = 0.25
_TOK_BLK = 256
_N_TOK = 8192


def _vq_tail_body(mask_ref, z_ref, mtok_ref, e_ref, idx_ref,
                  zqst_ref, loss_ref, ppl_ref, counts_ref, num_ref):
    i = pl.program_id(0)
    nsteps = pl.num_programs(0)

    e = e_ref[...]
    z = z_ref[...]
    mv = mtok_ref[...]
    idx = idx_ref[...]

    iota = lax.broadcasted_iota(jnp.int32, (_TOK_BLK, _N_E), 1)
    onehot = (iota == idx).astype(jnp.float32)

    zq = lax.dot_general(onehot, e, (((1,), (0,)), ((), ())),
                         preferred_element_type=jnp.float32,
                         precision=lax.Precision.HIGHEST)
    zqst_ref[...] = z + (zq - z)

    diff = zq - z
    part = jnp.sum(diff * diff * mv).reshape(1, 1)

    @pl.when(i == 0)
    def _init():
        counts_ref[...] = jnp.zeros_like(counts_ref)
        num_ref[...] = jnp.zeros_like(num_ref)

    counts_ref[...] += jnp.sum(onehot, axis=0, keepdims=True)
    num_ref[...] += part

    @pl.when(i == nsteps - 1)
    def _fini():
        msum = jnp.sum(mask_ref[...])
        denom = msum * (_N_TOK // 8) * _E_DIM
        loss_ref[...] = (1.0 + _BETA) * num_ref[...] / denom
        e_mean = counts_ref[...] / _N_TOK
        ent = jnp.sum(e_mean * jnp.log(e_mean + 1e-10)).reshape(1, 1)
        ppl_ref[...] = jnp.exp(-ent)


def kernel(z, mask, embedding_weight):
    z_p = jnp.transpose(z, (0, 2, 1))
    z_flat = z_p.reshape(_N_TOK, _E_DIM)
    mask_e = jnp.broadcast_to(mask[:, None, None], z_p.shape)
    mask_flat = mask_e.reshape(_N_TOK, _E_DIM)

    d = (jnp.sum(z_flat ** 2 * mask_flat, axis=1, keepdims=True)
         + jnp.sum(embedding_weight ** 2, axis=1)
         - 2.0 * jnp.matmul(z_flat * mask_flat, embedding_weight.T))
    mi = jnp.argmin(d, axis=1)
    enc = jnp.zeros((_N_TOK, _N_E), jnp.float32).at[
        jnp.arange(_N_TOK), mi].set(1.0)

    mask2d = mask[None, :]
    mask_tok = mask_flat[:, :1]
    mi2d = mi[:, None]

    nblk = _N_TOK // _TOK_BLK
    out_shape = (
        jax.ShapeDtypeStruct((_N_TOK, _E_DIM), jnp.float32),
        jax.ShapeDtypeStruct((1, 1), jnp.float32),
        jax.ShapeDtypeStruct((1, 1), jnp.float32),
        jax.ShapeDtypeStruct((1, _N_E), jnp.float32),
        jax.ShapeDtypeStruct((1, 1), jnp.float32),
    )
    grid_spec = pl.GridSpec(
        grid=(nblk,),
        in_specs=[
            pl.BlockSpec((1, 8), lambda i: (0, 0)),
            pl.BlockSpec((_TOK_BLK, _E_DIM), lambda i: (i, 0)),
            pl.BlockSpec((_TOK_BLK, 1), lambda i: (i, 0)),
            pl.BlockSpec((_N_E, _E_DIM), lambda i: (0, 0)),
            pl.BlockSpec((_TOK_BLK, 1), lambda i: (i, 0)),
        ],
        out_specs=[
            pl.BlockSpec((_TOK_BLK, _E_DIM), lambda i: (i, 0)),
            pl.BlockSpec((1, 1), lambda i: (0, 0)),
            pl.BlockSpec((1, 1), lambda i: (0, 0)),
            pl.BlockSpec((1, _N_E), lambda i: (0, 0)),
            pl.BlockSpec((1, 1), lambda i: (0, 0)),
        ],
    )
    zqst, loss, ppl, _counts, _num = pl.pallas_call(
        _vq_tail_body,
        grid_spec=grid_spec,
        out_shape=out_shape,
    )(mask2d, z_flat, mask_tok, embedding_weight, mi2d)

    z_q_out = jnp.transpose(zqst.reshape(z_p.shape), (0, 2, 1))
    return (loss[0, 0], z_q_out, ppl[0, 0], enc, mi2d)


# speedup vs baseline: 1.0919x; 1.0919x over previous
import jax
import jax.numpy as jnp
from jax import lax
from jax.experimental import pallas as pl

_N_E = 8192
_E_DIM = 32
_BETA = 0.25
_TOK_BLK = 256
_N_TOK = 8192


def _vq_tail_body(mask_ref, z_ref, mtok_ref, e_ref, idx_ref,
                  zqst_ref, loss_ref, ppl_ref, counts_ref, num_ref):
    i = pl.program_id(0)
    nsteps = pl.num_programs(0)

    e = e_ref[...]
    z = z_ref[...]
    mv = mtok_ref[...]
    idx = idx_ref[...]

    iota = lax.broadcasted_iota(jnp.int32, (_TOK_BLK, _N_E), 1)
    onehot = (iota == idx).astype(jnp.float32)

    zq = lax.dot_general(onehot, e, (((1,), (0,)), ((), ())),
                         preferred_element_type=jnp.float32)
    zqst_ref[...] = z + (zq - z)

    diff = zq - z
    part = jnp.sum(diff * diff * mv).reshape(1, 1)

    @pl.when(i == 0)
    def _init():
        counts_ref[...] = jnp.zeros_like(counts_ref)
        num_ref[...] = jnp.zeros_like(num_ref)

    ones_row = jnp.ones((1, _TOK_BLK), jnp.float32)
    counts_ref[...] += lax.dot_general(ones_row, onehot, (((1,), (0,)), ((), ())),
                                       preferred_element_type=jnp.float32)
    num_ref[...] += part

    @pl.when(i == nsteps - 1)
    def _fini():
        msum = jnp.sum(mask_ref[...])
        denom = msum * (_N_TOK // 8) * _E_DIM
        loss_ref[...] = (1.0 + _BETA) * num_ref[...] / denom
        e_mean = counts_ref[...] / _N_TOK
        ent = jnp.sum(e_mean * jnp.log(e_mean + 1e-10)).reshape(1, 1)
        ppl_ref[...] = jnp.exp(-ent)


def kernel(z, mask, embedding_weight):
    z_p = jnp.transpose(z, (0, 2, 1))
    z_flat = z_p.reshape(_N_TOK, _E_DIM)
    mask_e = jnp.broadcast_to(mask[:, None, None], z_p.shape)
    mask_flat = mask_e.reshape(_N_TOK, _E_DIM)

    d = (jnp.sum(z_flat ** 2 * mask_flat, axis=1, keepdims=True)
         + jnp.sum(embedding_weight ** 2, axis=1)
         - 2.0 * jnp.matmul(z_flat * mask_flat, embedding_weight.T))
    mi = jnp.argmin(d, axis=1)
    enc = jnp.zeros((_N_TOK, _N_E), jnp.float32).at[
        jnp.arange(_N_TOK), mi].set(1.0)

    mask2d = mask[None, :]
    mask_tok = mask_flat[:, :1]
    mi2d = mi[:, None]

    nblk = _N_TOK // _TOK_BLK
    out_shape = (
        jax.ShapeDtypeStruct((_N_TOK, _E_DIM), jnp.float32),
        jax.ShapeDtypeStruct((1, 1), jnp.float32),
        jax.ShapeDtypeStruct((1, 1), jnp.float32),
        jax.ShapeDtypeStruct((1, _N_E), jnp.float32),
        jax.ShapeDtypeStruct((1, 1), jnp.float32),
    )
    grid_spec = pl.GridSpec(
        grid=(nblk,),
        in_specs=[
            pl.BlockSpec((1, 8), lambda i: (0, 0)),
            pl.BlockSpec((_TOK_BLK, _E_DIM), lambda i: (i, 0)),
            pl.BlockSpec((_TOK_BLK, 1), lambda i: (i, 0)),
            pl.BlockSpec((_N_E, _E_DIM), lambda i: (0, 0)),
            pl.BlockSpec((_TOK_BLK, 1), lambda i: (i, 0)),
        ],
        out_specs=[
            pl.BlockSpec((_TOK_BLK, _E_DIM), lambda i: (i, 0)),
            pl.BlockSpec((1, 1), lambda i: (0, 0)),
            pl.BlockSpec((1, 1), lambda i: (0, 0)),
            pl.BlockSpec((1, _N_E), lambda i: (0, 0)),
            pl.BlockSpec((1, 1), lambda i: (0, 0)),
        ],
    )
    zqst, loss, ppl, _counts, _num = pl.pallas_call(
        _vq_tail_body,
        grid_spec=grid_spec,
        out_shape=out_shape,
    )(mask2d, z_flat, mask_tok, embedding_weight, mi2d)

    z_q_out = jnp.transpose(zqst.reshape(z_p.shape), (0, 2, 1))
    return (loss[0, 0], z_q_out, ppl[0, 0], enc, mi2d)


# speedup vs baseline: 1.1179x; 1.0238x over previous
import functools

import jax
import jax.numpy as jnp
from jax import lax
from jax.experimental import pallas as pl
from jax.experimental.pallas import tpu as pltpu
from jax.experimental.pallas import tpu_sc as plsc

_N_E = 8192
_E_DIM = 32
_BETA = 0.25
_TOK_BLK = 256
_N_TOK = 8192

_SC_INFO = plsc.get_sparse_core_info()
_NW = _SC_INFO.num_cores * _SC_INFO.num_subcores
_B_PER_W = _N_TOK // _NW


_GATHER_D = 128


def _sc_gather(table, idx):
    mesh = plsc.VectorSubcoreMesh(core_axis_name="c", subcore_axis_name="s")

    @functools.partial(
        pl.kernel, mesh=mesh,
        out_type=jax.ShapeDtypeStruct((_N_TOK, _GATHER_D), jnp.float32),
        scratch_types=[
            pltpu.VMEM((_B_PER_W,), jnp.int32),
            pltpu.VMEM((_B_PER_W, _GATHER_D), jnp.float32),
            pltpu.SemaphoreType.DMA,
        ],
    )
    def k(table_hbm, idx_hbm, out_hbm, idx_v, rows_v, sem):
        wid = lax.axis_index("s") * _SC_INFO.num_cores + lax.axis_index("c")
        base = wid * _B_PER_W
        pltpu.sync_copy(idx_hbm.at[pl.ds(base, _B_PER_W)], idx_v)
        pltpu.async_copy(table_hbm.at[idx_v], rows_v, sem).wait()
        pltpu.sync_copy(rows_v, out_hbm.at[pl.ds(base, _B_PER_W)])

    return k(table, idx)


def _vq_tail_body(mask_ref, z_ref, mtok_ref, zq_ref, idx_ref,
                  zqst_ref, loss_ref, ppl_ref, counts_ref, num_ref):
    i = pl.program_id(0)
    nsteps = pl.num_programs(0)

    z = z_ref[...]
    mv = mtok_ref[...]
    zq = zq_ref[...]
    idx = idx_ref[...]

    iota = lax.broadcasted_iota(jnp.int32, (_TOK_BLK, _N_E), 1)
    onehot = (iota == idx).astype(jnp.float32)

    zqst_ref[...] = z + (zq - z)

    diff = zq - z
    part = jnp.sum(diff * diff * mv).reshape(1, 1)

    @pl.when(i == 0)
    def _init():
        counts_ref[...] = jnp.zeros_like(counts_ref)
        num_ref[...] = jnp.zeros_like(num_ref)

    ones_row = jnp.ones((1, _TOK_BLK), jnp.float32)
    counts_ref[...] += lax.dot_general(ones_row, onehot, (((1,), (0,)), ((), ())),
                                       preferred_element_type=jnp.float32)
    num_ref[...] += part

    @pl.when(i == nsteps - 1)
    def _fini():
        msum = jnp.sum(mask_ref[...])
        denom = msum * (_N_TOK // 8) * _E_DIM
        loss_ref[...] = (1.0 + _BETA) * num_ref[...] / denom
        e_mean = counts_ref[...] / _N_TOK
        ent = jnp.sum(e_mean * jnp.log(e_mean + 1e-10)).reshape(1, 1)
        ppl_ref[...] = jnp.exp(-ent)


def kernel(z, mask, embedding_weight):
    z_p = jnp.transpose(z, (0, 2, 1))
    z_flat = z_p.reshape(_N_TOK, _E_DIM)
    mask_e = jnp.broadcast_to(mask[:, None, None], z_p.shape)
    mask_flat = mask_e.reshape(_N_TOK, _E_DIM)

    d = (jnp.sum(z_flat ** 2 * mask_flat, axis=1, keepdims=True)
         + jnp.sum(embedding_weight ** 2, axis=1)
         - 2.0 * jnp.matmul(z_flat * mask_flat, embedding_weight.T))
    mi = jnp.argmin(d, axis=1)
    enc = jnp.zeros((_N_TOK, _N_E), jnp.float32).at[
        jnp.arange(_N_TOK), mi].set(1.0)

    table = jnp.pad(embedding_weight, ((0, 0), (0, _GATHER_D - _E_DIM)))
    zq_flat = _sc_gather(table, mi)[:, :_E_DIM]

    mask2d = mask[None, :]
    mask_tok = mask_flat[:, :1]
    mi2d = mi[:, None]

    nblk = _N_TOK // _TOK_BLK
    out_shape = (
        jax.ShapeDtypeStruct((_N_TOK, _E_DIM), jnp.float32),
        jax.ShapeDtypeStruct((1, 1), jnp.float32),
        jax.ShapeDtypeStruct((1, 1), jnp.float32),
        jax.ShapeDtypeStruct((1, _N_E), jnp.float32),
        jax.ShapeDtypeStruct((1, 1), jnp.float32),
    )
    grid_spec = pl.GridSpec(
        grid=(nblk,),
        in_specs=[
            pl.BlockSpec((1, 8), lambda i: (0, 0)),
            pl.BlockSpec((_TOK_BLK, _E_DIM), lambda i: (i, 0)),
            pl.BlockSpec((_TOK_BLK, 1), lambda i: (i, 0)),
            pl.BlockSpec((_TOK_BLK, _E_DIM), lambda i: (i, 0)),
            pl.BlockSpec((_TOK_BLK, 1), lambda i: (i, 0)),
        ],
        out_specs=[
            pl.BlockSpec((_TOK_BLK, _E_DIM), lambda i: (i, 0)),
            pl.BlockSpec((1, 1), lambda i: (0, 0)),
            pl.BlockSpec((1, 1), lambda i: (0, 0)),
            pl.BlockSpec((1, _N_E), lambda i: (0, 0)),
            pl.BlockSpec((1, 1), lambda i: (0, 0)),
        ],
    )
    zqst, loss, ppl, _counts, _num = pl.pallas_call(
        _vq_tail_body,
        grid_spec=grid_spec,
        out_shape=out_shape,
    )(mask2d, z_flat, mask_tok, zq_flat, mi2d)

    z_q_out = jnp.transpose(zqst.reshape(z_p.shape), (0, 2, 1))
    return (loss[0, 0], z_q_out, ppl[0, 0], enc, mi2d)
